# speedup vs baseline: 3.2828x; 3.2828x over previous
import functools

import jax
import jax.numpy as jnp
from jax import lax
from jax.experimental import pallas as pl
from jax.experimental.pallas import tpu as pltpu
from jax.experimental.pallas import tpu_sc as plsc

VOCAB = 100000
EMBED = 128
BATCH = 4096
HIST = 50
B = BATCH * HIST

NC = 2
NS = 16
NW = NC * NS
B_PER_W = B // NW
CHUNK = 128
NCHUNK = B_PER_W // CHUNK
NBUF = 5
NGROUP = NCHUNK // NBUF

_mesh = plsc.VectorSubcoreMesh(core_axis_name="c", subcore_axis_name="s")


@functools.partial(
    pl.kernel,
    out_type=jax.ShapeDtypeStruct((B, EMBED), jnp.float32),
    mesh=_mesh,
    scratch_types=[
        pltpu.VMEM((NCHUNK, CHUNK), jnp.int32),
        pltpu.VMEM((NBUF, CHUNK, EMBED), jnp.float32),
        pltpu.SemaphoreType.DMA((NBUF,)),
        pltpu.SemaphoreType.DMA((NBUF,)),
    ],
)
def _sc_gather(idx_hbm, table_hbm, out_hbm, idx_v, rows_v, gsem, osem):
    wid = lax.axis_index("s") * NC + lax.axis_index("c")
    base = wid * B_PER_W
    pltpu.sync_copy(idx_hbm.at[wid], idx_v)

    @pl.loop(0, NGROUP)
    def _group(g):
        c0 = g * NBUF
        gathers = []
        for b in range(NBUF):
            gathers.append(
                pltpu.async_copy(
                    table_hbm.at[idx_v.at[c0 + b]], rows_v.at[b], gsem.at[b]
                )
            )
        writes = []
        for b in range(NBUF):
            gathers[b].wait()
            writes.append(
                pltpu.async_copy(
                    rows_v.at[b],
                    out_hbm.at[pl.ds(base + (c0 + b) * CHUNK, CHUNK)],
                    osem.at[b],
                )
            )
        for b in range(NBUF):
            writes[b].wait()


def kernel(x, table):
    idx = x.reshape(NW, NCHUNK, CHUNK).astype(jnp.int32)
    out = _sc_gather(idx, table)
    return out.reshape(BATCH, HIST, EMBED)


# speedup vs baseline: 5.8063x; 1.7687x over previous
import functools

import jax
import jax.numpy as jnp
from jax import lax
from jax.experimental import pallas as pl
from jax.experimental.pallas import tpu as pltpu
from jax.experimental.pallas import tpu_sc as plsc

VOCAB = 100000
EMBED = 128
BATCH = 4096
HIST = 50

NC = 2
NS = 16
NW = NC * NS
BT_PER_W = BATCH // NW
CB = 4
NCHUNK = BT_PER_W // CB
NBUF = 4
NGROUP = NCHUNK // NBUF

_mesh = plsc.VectorSubcoreMesh(core_axis_name="c", subcore_axis_name="s")


@functools.partial(
    pl.kernel,
    out_type=jax.ShapeDtypeStruct((BATCH, HIST, EMBED), jnp.float32),
    mesh=_mesh,
    scratch_types=[
        pltpu.VMEM((BT_PER_W, HIST), jnp.int32),
        pltpu.VMEM((NBUF, CB, HIST, EMBED), jnp.float32),
        pltpu.SemaphoreType.DMA((NBUF,)),
        pltpu.SemaphoreType.DMA((NBUF,)),
    ],
)
def _sc_gather(idx_hbm, table_hbm, out_hbm, idx_v, rows_v, gsem, osem):
    wid = lax.axis_index("s") * NC + lax.axis_index("c")
    base = wid * BT_PER_W
    pltpu.sync_copy(idx_hbm.at[pl.ds(base, BT_PER_W)], idx_v)

    @pl.loop(0, NGROUP)
    def _group(g):
        c0 = g * NBUF
        gathers = []
        for b in range(NBUF):
            for j in range(CB):
                gathers.append(
                    pltpu.async_copy(
                        table_hbm.at[idx_v.at[(c0 + b) * CB + j]],
                        rows_v.at[b, j],
                        gsem.at[b],
                    )
                )
        writes = []
        for b in range(NBUF):
            for j in range(CB):
                gathers[b * CB + j].wait()
            writes.append(
                pltpu.async_copy(
                    rows_v.at[b],
                    out_hbm.at[pl.ds(base + (c0 + b) * CB, CB)],
                    osem.at[b],
                )
            )
        for b in range(NBUF):
            writes[b].wait()


def kernel(x, table):
    idx = x.astype(jnp.int32)
    return _sc_gather(idx, table)


# speedup vs baseline: 5.8129x; 1.0011x over previous
import functools

import jax
import jax.numpy as jnp
from jax import lax
from jax.experimental import pallas as pl
from jax.experimental.pallas import tpu as pltpu
from jax.experimental.pallas import tpu_sc as plsc

VOCAB = 100000
EMBED = 128
BATCH = 4096
HIST = 50

NC = 2
NS = 16
NW = NC * NS
BT_PER_W = BATCH // NW
CB = 4
NCHUNK = BT_PER_W // CB
NBUF = 4
NGROUP = NCHUNK // NBUF

_mesh = plsc.VectorSubcoreMesh(core_axis_name="c", subcore_axis_name="s")


@functools.partial(
    pl.kernel,
    out_type=jax.ShapeDtypeStruct((BATCH, HIST, EMBED), jnp.float32),
    mesh=_mesh,
    scratch_types=[
        pltpu.VMEM((BT_PER_W, HIST), jnp.int32),
        pltpu.VMEM((NBUF, CB, HIST, EMBED), jnp.float32),
        pltpu.SemaphoreType.DMA((NBUF,)),
        pltpu.SemaphoreType.DMA((NBUF,)),
    ],
    compiler_params=pltpu.CompilerParams(use_tc_tiling_on_sc=True),
)
def _sc_gather(idx_hbm, table_hbm, out_hbm, idx_v, rows_v, gsem, osem):
    wid = lax.axis_index("s") * NC + lax.axis_index("c")
    base = wid * BT_PER_W
    pltpu.sync_copy(idx_hbm.at[pl.ds(base, BT_PER_W)], idx_v)

    @pl.loop(0, NGROUP)
    def _group(g):
        c0 = g * NBUF
        gathers = []
        for b in range(NBUF):
            for j in range(CB):
                gathers.append(
                    pltpu.async_copy(
                        table_hbm.at[idx_v.at[(c0 + b) * CB + j]],
                        rows_v.at[b, j],
                        gsem.at[b],
                    )
                )
        writes = []
        for b in range(NBUF):
            for j in range(CB):
                gathers[b * CB + j].wait()
            writes.append(
                pltpu.async_copy(
                    rows_v.at[b],
                    out_hbm.at[pl.ds(base + (c0 + b) * CB, CB)],
                    osem.at[b],
                )
            )
        for b in range(NBUF):
            writes[b].wait()


def kernel(x, table):
    idx = x.astype(jnp.int32)
    return _sc_gather(idx, table)


# speedup vs baseline: 9.7713x; 1.6810x over previous
import functools

import jax
import jax.numpy as jnp
from jax import lax
from jax.experimental import pallas as pl
from jax.experimental.pallas import tpu as pltpu
from jax.experimental.pallas import tpu_sc as plsc

VOCAB = 100000
EMBED = 128
BATCH = 4096
HIST = 50
B = BATCH * HIST

NC = 2
NS = 16
NW = NC * NS
B_PER_W = B // NW
CHUNK = 128
NCHUNK = B_PER_W // CHUNK
NBUF = 5
NGROUP = NCHUNK // NBUF

_mesh = plsc.VectorSubcoreMesh(core_axis_name="c", subcore_axis_name="s")


@functools.partial(
    pl.kernel,
    out_type=jax.ShapeDtypeStruct((B, EMBED), jnp.float32),
    mesh=_mesh,
    scratch_types=[
        pltpu.VMEM((NCHUNK, CHUNK), jnp.int32),
        pltpu.VMEM((NCHUNK, CHUNK), jnp.int32),
        pltpu.VMEM((NBUF, CHUNK, EMBED), jnp.float32),
        pltpu.SemaphoreType.DMA((NBUF,)),
        pltpu.SemaphoreType.DMA((NBUF,)),
    ],
)
def _sc_gather(idx_hbm, dst_hbm, table_hbm, out_hbm, idx_v, dst_v, rows_v,
               gsem, osem):
    wid = lax.axis_index("s") * NC + lax.axis_index("c")
    pltpu.sync_copy(idx_hbm.at[wid], idx_v)
    pltpu.sync_copy(dst_hbm.at[wid], dst_v)

    @pl.loop(0, NGROUP)
    def _group(g):
        c0 = g * NBUF
        gathers = []
        for b in range(NBUF):
            gathers.append(
                pltpu.async_copy(
                    table_hbm.at[idx_v.at[c0 + b]], rows_v.at[b], gsem.at[b]
                )
            )
        scatters = []
        for b in range(NBUF):
            gathers[b].wait()
            scatters.append(
                pltpu.async_copy(
                    rows_v.at[b], out_hbm.at[dst_v.at[c0 + b]], osem.at[b]
                )
            )
        for b in range(NBUF):
            scatters[b].wait()


def kernel(x, table):
    idx = x.reshape(NW, NCHUNK, CHUNK).astype(jnp.int32)
    p = jnp.arange(B, dtype=jnp.int32)
    dst = ((p % HIST) * BATCH + p // HIST).reshape(NW, NCHUNK, CHUNK)
    out = _sc_gather(idx, dst, table)
    return out.reshape(HIST, BATCH, EMBED).swapaxes(0, 1)


# speedup vs baseline: 10.2686x; 1.0509x over previous
import functools

import jax
import jax.numpy as jnp
from jax import lax
from jax.experimental import pallas as pl
from jax.experimental.pallas import tpu as pltpu
from jax.experimental.pallas import tpu_sc as plsc

VOCAB = 100000
EMBED = 128
BATCH = 4096
HIST = 50
B = BATCH * HIST

NC = 2
NS = 16
NW = NC * NS
B_PER_W = B // NW
CHUNK = 128
NCHUNK = B_PER_W // CHUNK
NBUF = 5
NGROUP = NCHUNK // NBUF

_mesh = plsc.VectorSubcoreMesh(core_axis_name="c", subcore_axis_name="s")


@functools.partial(
    pl.kernel,
    out_type=jax.ShapeDtypeStruct((B, EMBED), jnp.float32),
    mesh=_mesh,
    scratch_types=[
        pltpu.VMEM((NCHUNK, CHUNK), jnp.int32),
        pltpu.VMEM((NCHUNK, CHUNK), jnp.int32),
        pltpu.VMEM((NBUF, CHUNK, EMBED), jnp.float32),
        pltpu.SemaphoreType.DMA,
        pltpu.SemaphoreType.DMA((NBUF,)),
        pltpu.SemaphoreType.DMA((NBUF,)),
    ],
)
def _sc_gather(idx_hbm, dst_hbm, table_hbm, out_hbm, idx_v, dst_v, rows_v,
               ssem, gsem, osem):
    wid = lax.axis_index("s") * NC + lax.axis_index("c")
    s1 = pltpu.async_copy(idx_hbm.at[wid], idx_v, ssem)
    s2 = pltpu.async_copy(dst_hbm.at[wid], dst_v, ssem)
    s1.wait()
    s2.wait()

    def fire_gather(c, b):
        pltpu.async_copy(table_hbm.at[idx_v.at[c]], rows_v.at[b], gsem.at[b])

    def wait_gather(c, b):
        pltpu.make_async_copy(
            table_hbm.at[idx_v.at[c]], rows_v.at[b], gsem.at[b]
        ).wait()

    def fire_scatter(c, b):
        pltpu.async_copy(rows_v.at[b], out_hbm.at[dst_v.at[c]], osem.at[b])

    def wait_scatter(c, b):
        pltpu.make_async_copy(
            rows_v.at[b], out_hbm.at[dst_v.at[c]], osem.at[b]
        ).wait()

    for b in range(NBUF):
        fire_gather(b, b)

    @pl.loop(0, NGROUP - 1)
    def _group(g):
        c0 = g * NBUF
        for b in range(NBUF):
            c = c0 + b
            wait_gather(c, b)
            fire_scatter(c, b)
            wait_scatter(c, b)
            fire_gather(c + NBUF, b)
    c0 = (NGROUP - 1) * NBUF
    for b in range(NBUF):
        c = c0 + b
        wait_gather(c, b)
        fire_scatter(c, b)
    for b in range(NBUF):
        wait_scatter(c0 + b, b)


def kernel(x, table):
    idx = x.reshape(NW, NCHUNK, CHUNK).astype(jnp.int32)
    p = jnp.arange(B, dtype=jnp.int32)
    dst = ((p % HIST) * BATCH + p // HIST).reshape(NW, NCHUNK, CHUNK)
    out = _sc_gather(idx, dst, table)
    return out.reshape(HIST, BATCH, EMBED).swapaxes(0, 1)
